# Initial kernel scaffold; baseline (speedup 1.0000x reference)
import jax
import jax.numpy as jnp
from jax import lax
from jax.experimental import pallas as pl
from jax.experimental.pallas import tpu as pltpu
from jax.experimental.pallas import tpu_sc as plsc

N = 100000
E = 1600000
F = 16
H = 128
G = 256

NP = 102400
ER = 12512
EP = ER * 128
NCHUNK = 12800

_MESH = dict(core_axis_name="c", subcore_axis_name="s", num_cores=2,
             num_subcores=16)


def _zero_fill(zb, rows):
    @pl.loop(0, rows)
    def _(i):
        for k in range(8):
            zb[i, pl.ds(k * 16, 16)] = jnp.zeros((16,), jnp.float32)


def _deg_body(dstm, out, deg_sh, dsts_v, ones_v, zb):
    c = lax.axis_index("c")
    s = lax.axis_index("s")

    @pl.loop(0, 8)
    def _(k):
        ones_v[pl.ds(k * 16, 16)] = jnp.ones((16,), jnp.float32)

    @pl.loop(0, 400)
    def _(i):
        zb[pl.ds(i * 16, 16)] = jnp.zeros((16,), jnp.float32)

    pltpu.sync_copy(zb, deg_sh.at[pl.ds(s * 6400, 6400)])
    plsc.subcore_barrier()

    @pl.loop(0, 17)
    def _(b):
        rowbase = c * 6256 + s * 391 + b * 23
        pltpu.sync_copy(dstm.at[pl.ds(rowbase, 23), :], dsts_v)

        @pl.loop(0, 23)
        def _(j):
            pltpu.sync_copy(ones_v, deg_sh.at[dsts_v.at[j]], add=True)

    plsc.subcore_barrier()
    pltpu.sync_copy(deg_sh.at[pl.ds(s * 6400, 6400)],
                    out.at[c, pl.ds(s * 6400, 6400)])


_deg_call = pl.kernel(
    _deg_body,
    out_type=jax.ShapeDtypeStruct((2, NP), jnp.float32),
    mesh=plsc.VectorSubcoreMesh(**_MESH),
    scratch_types=[
        pltpu.VMEM_SHARED((NP,), jnp.float32),
        pltpu.VMEM((23, 128), jnp.int32),
        pltpu.VMEM((128,), jnp.float32),
        pltpu.VMEM((6400,), jnp.float32),
    ],
)


def _prop16_body(srcm, dstm, xs, out, acc_sh, srcs_v, dsts_v, rows_v, zb):
    c = lax.axis_index("c")
    s = lax.axis_index("s")

    _zero_fill(zb, 400)

    @pl.loop(0, 16)
    def _(k):
        pltpu.sync_copy(zb, acc_sh.at[pl.ds(s * 6400 + k * 400, 400), :])

    plsc.subcore_barrier()

    @pl.loop(0, 17)
    def _(b):
        rowbase = c * 6256 + s * 391 + b * 23
        pltpu.sync_copy(srcm.at[pl.ds(rowbase, 23), :], srcs_v)
        pltpu.sync_copy(dstm.at[pl.ds(rowbase, 23), :], dsts_v)

        @pl.loop(0, 23)
        def _(j):
            pltpu.sync_copy(xs.at[srcs_v.at[j]], rows_v)
            pltpu.sync_copy(rows_v, acc_sh.at[dsts_v.at[j]], add=True)

    plsc.subcore_barrier()

    @pl.loop(0, 16)
    def _(k):
        pltpu.sync_copy(acc_sh.at[pl.ds(s * 6400 + k * 400, 400), :],
                        out.at[c, pl.ds(s * 6400 + k * 400, 400), :])


_prop16_call = pl.kernel(
    _prop16_body,
    out_type=jax.ShapeDtypeStruct((2, NP, F), jnp.float32),
    mesh=plsc.VectorSubcoreMesh(**_MESH),
    scratch_types=[
        pltpu.VMEM_SHARED((NP, F), jnp.float32),
        pltpu.VMEM((23, 128), jnp.int32),
        pltpu.VMEM((23, 128), jnp.int32),
        pltpu.VMEM((128, F), jnp.float32),
        pltpu.VMEM((400, F), jnp.float32),
    ],
)


def _prop128_body(srcm, dstm, hs2, out, acc_sh, srcs_v, dsts_v, msrc_v,
                  mrel_v, rows_v, zb):
    c = lax.axis_index("c")
    s = lax.axis_index("s")

    _zero_fill(zb, 50)

    for p in range(4):
        base = (2 * p + c) * NCHUNK

        @pl.loop(0, 16)
        def _(k):
            pltpu.sync_copy(zb, acc_sh.at[pl.ds(s * 800 + k * 50, 50), :])

        plsc.subcore_barrier()

        @pl.loop(0, 34)
        def _(b):
            rowbase = s * 782 + b * 23
            pltpu.sync_copy(srcm.at[pl.ds(rowbase, 23), :], srcs_v)
            pltpu.sync_copy(dstm.at[pl.ds(rowbase, 23), :], dsts_v)

            @pl.loop(0, 23)
            def _(j):
                for k in range(8):
                    sl = pl.ds(k * 16, 16)
                    dv = dsts_v[j, sl]
                    sv = srcs_v[j, sl]
                    rel = dv - base
                    m = (rel >= 0) & (rel < NCHUNK)
                    mrel_v[j, sl] = jnp.where(m, rel, -1)
                    msrc_v[j, sl] = jnp.where(m, sv, -1)

            @pl.loop(0, 23)
            def _(j):
                pltpu.sync_copy(
                    hs2.at[plsc.Indices(msrc_v.at[j], ignored_value=-1)],
                    rows_v)
                pltpu.sync_copy(
                    rows_v,
                    acc_sh.at[plsc.Indices(mrel_v.at[j], ignored_value=-1)],
                    add=True)

        plsc.subcore_barrier()

        @pl.loop(0, 16)
        def _(k):
            pltpu.sync_copy(
                acc_sh.at[pl.ds(s * 800 + k * 50, 50), :],
                out.at[pl.ds(base + s * 800 + k * 50, 50), :])

        plsc.subcore_barrier()


_prop128_call = pl.kernel(
    _prop128_body,
    out_type=jax.ShapeDtypeStruct((NP, H), jnp.float32),
    mesh=plsc.VectorSubcoreMesh(**_MESH),
    scratch_types=[
        pltpu.VMEM_SHARED((NCHUNK, H), jnp.float32),
        pltpu.VMEM((23, 128), jnp.int32),
        pltpu.VMEM((23, 128), jnp.int32),
        pltpu.VMEM((23, 128), jnp.int32),
        pltpu.VMEM((23, 128), jnp.int32),
        pltpu.VMEM((128, H), jnp.float32),
        pltpu.VMEM((50, 128), jnp.float32),
    ],
)


def _scale_body(d0, d1, x, dinv_ref, xs_ref):
    dv = lax.rsqrt(d0[...] + d1[...] + 1.0)
    dinv_ref[...] = dv
    xs_ref[...] = x[...] * dv


def _scale_call(d0, d1, x):
    blk = 2000
    grid = N // blk
    return pl.pallas_call(
        _scale_body,
        grid=(grid,),
        in_specs=[
            pl.BlockSpec((blk, 1), lambda i: (i, 0)),
            pl.BlockSpec((blk, 1), lambda i: (i, 0)),
            pl.BlockSpec((blk, F), lambda i: (i, 0)),
        ],
        out_specs=[
            pl.BlockSpec((blk, 1), lambda i: (i, 0)),
            pl.BlockSpec((blk, F), lambda i: (i, 0)),
        ],
        out_shape=[
            jax.ShapeDtypeStruct((N, 1), jnp.float32),
            jax.ShapeDtypeStruct((N, F), jnp.float32),
        ],
    )(d0, d1, x)


def _mid_body(a0, a1, xs, dinv, W1, b1, W2, hs2_ref):
    ax = (a0[...] + a1[...] + xs[...]) * dinv[...]
    h1 = jnp.maximum(
        jnp.dot(ax, W1[...], preferred_element_type=jnp.float32) + b1[...],
        0.0)
    t2 = jnp.dot(h1, W2[...], preferred_element_type=jnp.float32)
    hs2_ref[...] = t2 * dinv[...]


def _mid_call(a0, a1, xs, dinv, W1, b1, W2):
    blk = 1000
    grid = N // blk
    return pl.pallas_call(
        _mid_body,
        grid=(grid,),
        in_specs=[
            pl.BlockSpec((blk, F), lambda i: (i, 0)),
            pl.BlockSpec((blk, F), lambda i: (i, 0)),
            pl.BlockSpec((blk, F), lambda i: (i, 0)),
            pl.BlockSpec((blk, 1), lambda i: (i, 0)),
            pl.BlockSpec((F, H), lambda i: (0, 0)),
            pl.BlockSpec((1, H), lambda i: (0, 0)),
            pl.BlockSpec((H, H), lambda i: (0, 0)),
        ],
        out_specs=pl.BlockSpec((blk, H), lambda i: (i, 0)),
        out_shape=jax.ShapeDtypeStruct((N, H), jnp.float32),
    )(a0, a1, xs, dinv, W1, b1, W2)


def _final_body(hs2, a2, dinv, b2, batch, Wlin, blin, out_ref, accp, cnt):
    i = pl.program_id(0)

    @pl.when(i == 0)
    def _():
        accp[...] = jnp.zeros_like(accp)
        cnt[...] = jnp.zeros_like(cnt)

    h2 = jnp.maximum((a2[...] + hs2[...]) * dinv[...] + b2[...], 0.0)
    b = jnp.reshape(batch[...], (1, batch.shape[-1]))
    oh = (lax.broadcasted_iota(jnp.int32, (G, b.shape[1]), 0) == b
          ).astype(jnp.float32)
    accp[...] += jnp.dot(oh, h2, preferred_element_type=jnp.float32)
    cnt[...] += jnp.sum(oh, axis=1, keepdims=True)

    @pl.when(i == pl.num_programs(0) - 1)
    def _():
        pooled = accp[...] / jnp.maximum(cnt[...], 1.0)
        out_ref[...] = (
            jnp.dot(pooled, Wlin[...], preferred_element_type=jnp.float32)
            + blin[...])


def _final_call(hs2, a2, dinv, b2, batchr, Wlin, blin):
    blk = 1000
    grid = N // blk
    return pl.pallas_call(
        _final_body,
        grid=(grid,),
        in_specs=[
            pl.BlockSpec((blk, H), lambda i: (i, 0)),
            pl.BlockSpec((blk, H), lambda i: (i, 0)),
            pl.BlockSpec((blk, 1), lambda i: (i, 0)),
            pl.BlockSpec((1, H), lambda i: (0, 0)),
            pl.BlockSpec((1, 1, blk), lambda i: (i, 0, 0)),
            pl.BlockSpec((H, 1), lambda i: (0, 0)),
            pl.BlockSpec((1, 1), lambda i: (0, 0)),
        ],
        out_specs=pl.BlockSpec((G, 1), lambda i: (0, 0)),
        out_shape=jax.ShapeDtypeStruct((G, 1), jnp.float32),
        scratch_shapes=[
            pltpu.VMEM((G, H), jnp.float32),
            pltpu.VMEM((G, 1), jnp.float32),
        ],
        compiler_params=pltpu.CompilerParams(
            dimension_semantics=("arbitrary",)),
    )(hs2, a2, dinv, b2, batchr, Wlin, blin)


def kernel(x, edge_index, batch, W1, b1, W2, b2, Wlin, blin):
    pad = EP - E
    lane = lax.iota(jnp.int32, pad)
    src_p = jnp.concatenate([edge_index[0], lane % 128])
    dst_p = jnp.concatenate([edge_index[1], N + (lane % 16)])
    srcm = src_p.reshape(ER, 128)
    dstm = dst_p.reshape(ER, 128)

    deg2 = _deg_call(dstm)
    dinv, xs = _scale_call(deg2[0, :N, None], deg2[1, :N, None], x)

    o16 = _prop16_call(srcm, dstm, xs)
    hs2 = _mid_call(o16[0, :N], o16[1, :N], xs, dinv, W1,
                    b1.reshape(1, H), W2)

    a2 = _prop128_call(srcm, dstm, hs2)
    out = _final_call(hs2, a2[:N], dinv, b2.reshape(1, H),
                      batch.reshape(100, 1, 1000), Wlin,
                      blin.reshape(1, 1))
    return out[:, 0]


# speedup vs baseline: 6.3159x; 6.3159x over previous
import jax
import jax.numpy as jnp
from jax import lax
from jax.experimental import pallas as pl
from jax.experimental.pallas import tpu as pltpu
from jax.experimental.pallas import tpu_sc as plsc

N = 100000
E = 1600000
F = 16
H = 128
G = 256

NP = 100352
ER = 12544
EP = ER * 128
NCHUNK = 12544
PT = NP // 16

_MESH = dict(core_axis_name="c", subcore_axis_name="s", num_cores=2,
             num_subcores=16)


def _zero_fill(zb):
    rows, cols = zb.shape

    @pl.loop(0, rows)
    def _(i):
        for k in range(cols // 16):
            zb[i, pl.ds(k * 16, 16)] = jnp.zeros((16,), jnp.float32)


def _deg_body(dstm, out, deg_sh, dsts_v, ones_v, zb):
    c = lax.axis_index("c")
    s = lax.axis_index("s")

    @pl.loop(0, 8)
    def _(k):
        ones_v[pl.ds(k * 16, 16)] = jnp.ones((16,), jnp.float32)

    @pl.loop(0, 392)
    def _(i):
        zb[pl.ds(i * 16, 16)] = jnp.zeros((16,), jnp.float32)

    pltpu.sync_copy(zb, deg_sh.at[pl.ds(s * PT, PT)])
    plsc.subcore_barrier()

    @pl.loop(0, 7)
    def _(b):
        rowbase = c * 6272 + s * 392 + b * 56
        pltpu.sync_copy(dstm.at[pl.ds(rowbase, 56), :], dsts_v)

        @pl.loop(0, 56)
        def _(j):
            pltpu.sync_copy(ones_v, deg_sh.at[dsts_v.at[j]], add=True)

    plsc.subcore_barrier()
    pltpu.sync_copy(deg_sh.at[pl.ds(s * PT, PT)],
                    out.at[c, pl.ds(s * PT, PT)])


_deg_call = pl.kernel(
    _deg_body,
    out_type=jax.ShapeDtypeStruct((2, NP), jnp.float32),
    mesh=plsc.VectorSubcoreMesh(**_MESH),
    scratch_types=[
        pltpu.VMEM_SHARED((NP,), jnp.float32),
        pltpu.VMEM((56, 128), jnp.int32),
        pltpu.VMEM((128,), jnp.float32),
        pltpu.VMEM((PT,), jnp.float32),
    ],
)


def _prop16_body(srcm, dstm, xs, out, acc_sh, srcs_v, dsts_v, rows_v, zb):
    c = lax.axis_index("c")
    s = lax.axis_index("s")

    _zero_fill(zb)

    @pl.loop(0, 98)
    def _(k):
        pltpu.sync_copy(zb, acc_sh.at[pl.ds(s * PT + k * 64, 64), :])

    plsc.subcore_barrier()

    @pl.loop(0, 7)
    def _(b):
        rowbase = c * 6272 + s * 392 + b * 56
        pltpu.sync_copy(srcm.at[pl.ds(rowbase, 56), :], srcs_v)
        pltpu.sync_copy(dstm.at[pl.ds(rowbase, 56), :], dsts_v)

        @pl.loop(0, 56)
        def _(j):
            pltpu.sync_copy(xs.at[srcs_v.at[j]], rows_v)
            pltpu.sync_copy(rows_v, acc_sh.at[dsts_v.at[j]], add=True)

    plsc.subcore_barrier()
    pltpu.sync_copy(acc_sh.at[pl.ds(s * PT, PT), :],
                    out.at[c, pl.ds(s * PT, PT), :])


_prop16_call = pl.kernel(
    _prop16_body,
    out_type=jax.ShapeDtypeStruct((2, NP, F), jnp.float32),
    mesh=plsc.VectorSubcoreMesh(**_MESH),
    compiler_params=pltpu.CompilerParams(use_tc_tiling_on_sc=False),
    scratch_types=[
        pltpu.VMEM_SHARED((NP, F), jnp.float32),
        pltpu.VMEM((56, 128), jnp.int32),
        pltpu.VMEM((56, 128), jnp.int32),
        pltpu.VMEM((128, F), jnp.float32),
        pltpu.VMEM((64, F), jnp.float32),
    ],
)


def _prop128_body(srcm, dstm, hs2, out, acc_sh, srcs_v, dsts_v, msrc_v,
                  mrel_v, rows_v, zb):
    c = lax.axis_index("c")
    s = lax.axis_index("s")

    _zero_fill(zb)

    for p in range(4):
        base = (2 * p + c) * NCHUNK

        @pl.loop(0, 49)
        def _(k):
            pltpu.sync_copy(zb, acc_sh.at[pl.ds(s * 784 + k * 16, 16), :])

        plsc.subcore_barrier()

        @pl.loop(0, 49)
        def _(b):
            rowbase = s * 784 + b * 16
            pltpu.sync_copy(srcm.at[pl.ds(rowbase, 16), :], srcs_v)
            pltpu.sync_copy(dstm.at[pl.ds(rowbase, 16), :], dsts_v)

            @pl.loop(0, 16)
            def _(j):
                for k in range(8):
                    sl = pl.ds(k * 16, 16)
                    dv = dsts_v[j, sl]
                    sv = srcs_v[j, sl]
                    rel = dv - base
                    m = (rel >= 0) & (rel < NCHUNK)
                    mrel_v[pl.ds(k * 16, 16)] = jnp.where(m, rel, -1)
                    msrc_v[pl.ds(k * 16, 16)] = jnp.where(m, sv, -1)

                pltpu.sync_copy(
                    hs2.at[plsc.Indices(msrc_v, ignored_value=-1)],
                    rows_v)
                pltpu.sync_copy(
                    rows_v,
                    acc_sh.at[plsc.Indices(mrel_v, ignored_value=-1)],
                    add=True)

        plsc.subcore_barrier()
        pltpu.sync_copy(acc_sh.at[pl.ds(s * 784, 784), :],
                        out.at[pl.ds(base + s * 784, 784), :])
        plsc.subcore_barrier()


_prop128_call = pl.kernel(
    _prop128_body,
    out_type=jax.ShapeDtypeStruct((NP, H), jnp.float32),
    mesh=plsc.VectorSubcoreMesh(**_MESH),
    scratch_types=[
        pltpu.VMEM_SHARED((NCHUNK, H), jnp.float32),
        pltpu.VMEM((16, 128), jnp.int32),
        pltpu.VMEM((16, 128), jnp.int32),
        pltpu.VMEM((128,), jnp.int32),
        pltpu.VMEM((128,), jnp.int32),
        pltpu.VMEM((128, H), jnp.float32),
        pltpu.VMEM((16, 128), jnp.float32),
    ],
)


def _scale_body(d0, d1, x, dinv_ref, xs_ref):
    dv = lax.rsqrt(d0[...] + d1[...] + 1.0)
    dinv_ref[...] = dv
    xs_ref[...] = x[...] * dv


def _scale_call(d0, d1, x):
    blk = 2000
    grid = N // blk
    return pl.pallas_call(
        _scale_body,
        grid=(grid,),
        in_specs=[
            pl.BlockSpec((blk, 1), lambda i: (i, 0)),
            pl.BlockSpec((blk, 1), lambda i: (i, 0)),
            pl.BlockSpec((blk, F), lambda i: (i, 0)),
        ],
        out_specs=[
            pl.BlockSpec((blk, 1), lambda i: (i, 0)),
            pl.BlockSpec((blk, F), lambda i: (i, 0)),
        ],
        out_shape=[
            jax.ShapeDtypeStruct((N, 1), jnp.float32),
            jax.ShapeDtypeStruct((N, F), jnp.float32),
        ],
    )(d0, d1, x)


def _mid_body(a0, a1, xs, dinv, W1, b1, W2, hs2_ref):
    ax = (a0[...] + a1[...] + xs[...]) * dinv[...]
    h1 = jnp.maximum(
        jnp.dot(ax, W1[...], preferred_element_type=jnp.float32) + b1[...],
        0.0)
    t2 = jnp.dot(h1, W2[...], preferred_element_type=jnp.float32)
    hs2_ref[...] = t2 * dinv[...]


def _mid_call(a0, a1, xs, dinv, W1, b1, W2):
    blk = 1000
    grid = N // blk
    return pl.pallas_call(
        _mid_body,
        grid=(grid,),
        in_specs=[
            pl.BlockSpec((blk, F), lambda i: (i, 0)),
            pl.BlockSpec((blk, F), lambda i: (i, 0)),
            pl.BlockSpec((blk, F), lambda i: (i, 0)),
            pl.BlockSpec((blk, 1), lambda i: (i, 0)),
            pl.BlockSpec((F, H), lambda i: (0, 0)),
            pl.BlockSpec((1, H), lambda i: (0, 0)),
            pl.BlockSpec((H, H), lambda i: (0, 0)),
        ],
        out_specs=pl.BlockSpec((blk, H), lambda i: (i, 0)),
        out_shape=jax.ShapeDtypeStruct((N, H), jnp.float32),
    )(a0, a1, xs, dinv, W1, b1, W2)


def _final_body(hs2, a2, dinv, b2, batch, Wlin, blin, out_ref, accp, cnt):
    i = pl.program_id(0)

    @pl.when(i == 0)
    def _():
        accp[...] = jnp.zeros_like(accp)
        cnt[...] = jnp.zeros_like(cnt)

    h2 = jnp.maximum((a2[...] + hs2[...]) * dinv[...] + b2[...], 0.0)
    b = jnp.reshape(batch[...], (1, batch.shape[-1]))
    oh = (lax.broadcasted_iota(jnp.int32, (G, b.shape[1]), 0) == b
          ).astype(jnp.float32)
    accp[...] += jnp.dot(oh, h2, preferred_element_type=jnp.float32)
    cnt[...] += jnp.sum(oh, axis=1, keepdims=True)

    @pl.when(i == pl.num_programs(0) - 1)
    def _():
        pooled = accp[...] / jnp.maximum(cnt[...], 1.0)
        out_ref[...] = (
            jnp.dot(pooled, Wlin[...], preferred_element_type=jnp.float32)
            + blin[...])


def _final_call(hs2, a2, dinv, b2, batchr, Wlin, blin):
    blk = 1000
    grid = N // blk
    return pl.pallas_call(
        _final_body,
        grid=(grid,),
        in_specs=[
            pl.BlockSpec((blk, H), lambda i: (i, 0)),
            pl.BlockSpec((blk, H), lambda i: (i, 0)),
            pl.BlockSpec((blk, 1), lambda i: (i, 0)),
            pl.BlockSpec((1, H), lambda i: (0, 0)),
            pl.BlockSpec((1, 1, blk), lambda i: (i, 0, 0)),
            pl.BlockSpec((H, 1), lambda i: (0, 0)),
            pl.BlockSpec((1, 1), lambda i: (0, 0)),
        ],
        out_specs=pl.BlockSpec((G, 1), lambda i: (0, 0)),
        out_shape=jax.ShapeDtypeStruct((G, 1), jnp.float32),
        scratch_shapes=[
            pltpu.VMEM((G, H), jnp.float32),
            pltpu.VMEM((G, 1), jnp.float32),
        ],
        compiler_params=pltpu.CompilerParams(
            dimension_semantics=("arbitrary",)),
    )(hs2, a2, dinv, b2, batchr, Wlin, blin)


def kernel(x, edge_index, batch, W1, b1, W2, b2, Wlin, blin):
    pad = EP - E
    lane = lax.iota(jnp.int32, pad)
    src_p = jnp.concatenate([edge_index[0], lane % 128])
    dst_p = jnp.concatenate([edge_index[1], N + (lane % 16)])
    srcm = src_p.reshape(ER, 128)
    dstm = dst_p.reshape(ER, 128)

    deg2 = _deg_call(dstm)
    dinv, xs = _scale_call(deg2[0, :N, None], deg2[1, :N, None], x)

    o16 = _prop16_call(srcm, dstm, xs)
    hs2 = _mid_call(o16[0, :N], o16[1, :N], xs, dinv, W1,
                    b1.reshape(1, H), W2)

    a2 = _prop128_call(srcm, dstm, hs2)
    out = _final_call(hs2, a2[:N], dinv, b2.reshape(1, H),
                      batch.reshape(100, 1, 1000), Wlin,
                      blin.reshape(1, 1))
    return out[:, 0]


# speedup vs baseline: 8.5726x; 1.3573x over previous
import jax
import jax.numpy as jnp
from jax import lax
from jax.experimental import pallas as pl
from jax.experimental.pallas import tpu as pltpu
from jax.experimental.pallas import tpu_sc as plsc

N = 100000
E = 1600000
F = 16
H = 128
G = 256

NP = 100352
ER = 12544
EP = ER * 128
NCHUNK = 12544
PT = NP // 16

_MESH = dict(core_axis_name="c", subcore_axis_name="s", num_cores=2,
             num_subcores=16)


def _zero_fill(zb):
    rows, cols = zb.shape

    @pl.loop(0, rows)
    def _(i):
        for k in range(cols // 16):
            zb[i, pl.ds(k * 16, 16)] = jnp.zeros((16,), jnp.float32)


def _deg_body(dstm, out, deg_sh, dsts_v, ones_v, zb):
    c = lax.axis_index("c")
    s = lax.axis_index("s")

    @pl.loop(0, 8)
    def _(k):
        ones_v[pl.ds(k * 16, 16)] = jnp.ones((16,), jnp.float32)

    @pl.loop(0, 392)
    def _(i):
        zb[pl.ds(i * 16, 16)] = jnp.zeros((16,), jnp.float32)

    pltpu.sync_copy(zb, deg_sh.at[pl.ds(s * PT, PT)])
    plsc.subcore_barrier()

    @pl.loop(0, 7)
    def _(b):
        rowbase = c * 6272 + s * 392 + b * 56
        pltpu.sync_copy(dstm.at[pl.ds(rowbase, 56), :], dsts_v)

        @pl.loop(0, 56)
        def _(j):
            pltpu.sync_copy(ones_v, deg_sh.at[dsts_v.at[j]], add=True)

    plsc.subcore_barrier()
    pltpu.sync_copy(deg_sh.at[pl.ds(s * PT, PT)],
                    out.at[c, pl.ds(s * PT, PT)])


_deg_call = pl.kernel(
    _deg_body,
    out_type=jax.ShapeDtypeStruct((2, NP), jnp.float32),
    mesh=plsc.VectorSubcoreMesh(**_MESH),
    scratch_types=[
        pltpu.VMEM_SHARED((NP,), jnp.float32),
        pltpu.VMEM((56, 128), jnp.int32),
        pltpu.VMEM((128,), jnp.float32),
        pltpu.VMEM((PT,), jnp.float32),
    ],
)


def _prop16_body(srcm, dstm, xs, out, acc_sh, srcs_v, dsts_v, rows_v, zb):
    c = lax.axis_index("c")
    s = lax.axis_index("s")

    _zero_fill(zb)

    @pl.loop(0, 98)
    def _(k):
        pltpu.sync_copy(zb, acc_sh.at[pl.ds(s * PT + k * 64, 64), :])

    plsc.subcore_barrier()

    @pl.loop(0, 7)
    def _(b):
        rowbase = c * 6272 + s * 392 + b * 56
        pltpu.sync_copy(srcm.at[pl.ds(rowbase, 56), :], srcs_v)
        pltpu.sync_copy(dstm.at[pl.ds(rowbase, 56), :], dsts_v)

        @pl.loop(0, 56)
        def _(j):
            pltpu.sync_copy(xs.at[srcs_v.at[j]], rows_v)
            pltpu.sync_copy(rows_v, acc_sh.at[dsts_v.at[j]], add=True)

    plsc.subcore_barrier()
    pltpu.sync_copy(acc_sh.at[pl.ds(s * PT, PT), :],
                    out.at[c, pl.ds(s * PT, PT), :])


_prop16_call = pl.kernel(
    _prop16_body,
    out_type=jax.ShapeDtypeStruct((2, NP, F), jnp.float32),
    mesh=plsc.VectorSubcoreMesh(**_MESH),
    compiler_params=pltpu.CompilerParams(use_tc_tiling_on_sc=False),
    scratch_types=[
        pltpu.VMEM_SHARED((NP, F), jnp.float32),
        pltpu.VMEM((56, 128), jnp.int32),
        pltpu.VMEM((56, 128), jnp.int32),
        pltpu.VMEM((128, F), jnp.float32),
        pltpu.VMEM((64, F), jnp.float32),
    ],
)


def _prop128_body(srcm, dstm, hs2, out, acc_sh, srcs_v, dsts_v, msrc_v,
                  mrel_v, rows_v, zb, gsem, ssem):
    c = lax.axis_index("c")
    s = lax.axis_index("s")

    _zero_fill(zb)

    for p in range(4):
        base = (2 * p + c) * NCHUNK

        @pl.loop(0, 49)
        def _(k):
            pltpu.sync_copy(zb, acc_sh.at[pl.ds(s * 784 + k * 16, 16), :])

        plsc.subcore_barrier()

        @pl.loop(0, 49)
        def _(b):
            rowbase = s * 784 + b * 16
            pltpu.sync_copy(srcm.at[pl.ds(rowbase, 16), :], srcs_v)
            pltpu.sync_copy(dstm.at[pl.ds(rowbase, 16), :], dsts_v)

            def masks(h):
                j, half = h // 2, h % 2
                for k in range(4):
                    sl = pl.ds(half * 64 + k * 16, 16)
                    so = pl.ds(k * 16, 16)
                    rel = dsts_v[j, sl] - base
                    m = (rel >= 0) & (rel < NCHUNK)
                    mrel_v[h % 2, so] = jnp.where(m, rel, -1)
                    msrc_v[h % 2, so] = jnp.where(m, srcs_v[j, sl], -1)

            def gather(h):
                return pltpu.async_copy(
                    hs2.at[plsc.Indices(msrc_v.at[h % 2], ignored_value=-1)],
                    rows_v.at[h % 2], gsem)

            def scatter(h):
                return pltpu.async_copy(
                    rows_v.at[h % 2],
                    acc_sh.at[plsc.Indices(mrel_v.at[h % 2],
                                           ignored_value=-1)],
                    ssem, add=True)

            nh = 32
            gd = [None] * nh
            sd = [None] * nh
            for h in range(nh):
                if h >= 2:
                    sd[h - 2].wait()
                masks(h)
                gd[h] = gather(h)
                if h >= 1:
                    gd[h - 1].wait()
                    sd[h - 1] = scatter(h - 1)
            sd[nh - 2].wait()
            gd[nh - 1].wait()
            sd[nh - 1] = scatter(nh - 1)
            sd[nh - 1].wait()

        plsc.subcore_barrier()
        pltpu.sync_copy(acc_sh.at[pl.ds(s * 784, 784), :],
                        out.at[pl.ds(base + s * 784, 784), :])
        plsc.subcore_barrier()


_prop128_call = pl.kernel(
    _prop128_body,
    out_type=jax.ShapeDtypeStruct((NP, H), jnp.float32),
    mesh=plsc.VectorSubcoreMesh(**_MESH),
    scratch_types=[
        pltpu.VMEM_SHARED((NCHUNK, H), jnp.float32),
        pltpu.VMEM((16, 128), jnp.int32),
        pltpu.VMEM((16, 128), jnp.int32),
        pltpu.VMEM((2, 64), jnp.int32),
        pltpu.VMEM((2, 64), jnp.int32),
        pltpu.VMEM((2, 64, H), jnp.float32),
        pltpu.VMEM((16, 128), jnp.float32),
        pltpu.SemaphoreType.DMA,
        pltpu.SemaphoreType.DMA,
    ],
)


def _scale_body(d0, d1, x, dinv_ref, xs_ref):
    dv = lax.rsqrt(d0[...] + d1[...] + 1.0)
    dinv_ref[...] = dv
    xs_ref[...] = x[...] * dv


def _scale_call(d0, d1, x):
    blk = 2000
    grid = N // blk
    return pl.pallas_call(
        _scale_body,
        grid=(grid,),
        in_specs=[
            pl.BlockSpec((blk, 1), lambda i: (i, 0)),
            pl.BlockSpec((blk, 1), lambda i: (i, 0)),
            pl.BlockSpec((blk, F), lambda i: (i, 0)),
        ],
        out_specs=[
            pl.BlockSpec((blk, 1), lambda i: (i, 0)),
            pl.BlockSpec((blk, F), lambda i: (i, 0)),
        ],
        out_shape=[
            jax.ShapeDtypeStruct((N, 1), jnp.float32),
            jax.ShapeDtypeStruct((N, F), jnp.float32),
        ],
    )(d0, d1, x)


def _mid_body(a0, a1, xs, dinv, W1, b1, W2, hs2_ref):
    ax = (a0[...] + a1[...] + xs[...]) * dinv[...]
    h1 = jnp.maximum(
        jnp.dot(ax, W1[...], preferred_element_type=jnp.float32) + b1[...],
        0.0)
    t2 = jnp.dot(h1, W2[...], preferred_element_type=jnp.float32)
    hs2_ref[...] = t2 * dinv[...]


def _mid_call(a0, a1, xs, dinv, W1, b1, W2):
    blk = 1000
    grid = N // blk
    return pl.pallas_call(
        _mid_body,
        grid=(grid,),
        in_specs=[
            pl.BlockSpec((blk, F), lambda i: (i, 0)),
            pl.BlockSpec((blk, F), lambda i: (i, 0)),
            pl.BlockSpec((blk, F), lambda i: (i, 0)),
            pl.BlockSpec((blk, 1), lambda i: (i, 0)),
            pl.BlockSpec((F, H), lambda i: (0, 0)),
            pl.BlockSpec((1, H), lambda i: (0, 0)),
            pl.BlockSpec((H, H), lambda i: (0, 0)),
        ],
        out_specs=pl.BlockSpec((blk, H), lambda i: (i, 0)),
        out_shape=jax.ShapeDtypeStruct((N, H), jnp.float32),
    )(a0, a1, xs, dinv, W1, b1, W2)


def _final_body(hs2, a2, dinv, b2, batch, Wlin, blin, out_ref, accp, cnt):
    i = pl.program_id(0)

    @pl.when(i == 0)
    def _():
        accp[...] = jnp.zeros_like(accp)
        cnt[...] = jnp.zeros_like(cnt)

    h2 = jnp.maximum((a2[...] + hs2[...]) * dinv[...] + b2[...], 0.0)
    b = jnp.reshape(batch[...], (1, batch.shape[-1]))
    oh = (lax.broadcasted_iota(jnp.int32, (G, b.shape[1]), 0) == b
          ).astype(jnp.float32)
    accp[...] += jnp.dot(oh, h2, preferred_element_type=jnp.float32)
    cnt[...] += jnp.sum(oh, axis=1, keepdims=True)

    @pl.when(i == pl.num_programs(0) - 1)
    def _():
        pooled = accp[...] / jnp.maximum(cnt[...], 1.0)
        out_ref[...] = (
            jnp.dot(pooled, Wlin[...], preferred_element_type=jnp.float32)
            + blin[...])


def _final_call(hs2, a2, dinv, b2, batchr, Wlin, blin):
    blk = 1000
    grid = N // blk
    return pl.pallas_call(
        _final_body,
        grid=(grid,),
        in_specs=[
            pl.BlockSpec((blk, H), lambda i: (i, 0)),
            pl.BlockSpec((blk, H), lambda i: (i, 0)),
            pl.BlockSpec((blk, 1), lambda i: (i, 0)),
            pl.BlockSpec((1, H), lambda i: (0, 0)),
            pl.BlockSpec((1, 1, blk), lambda i: (i, 0, 0)),
            pl.BlockSpec((H, 1), lambda i: (0, 0)),
            pl.BlockSpec((1, 1), lambda i: (0, 0)),
        ],
        out_specs=pl.BlockSpec((G, 1), lambda i: (0, 0)),
        out_shape=jax.ShapeDtypeStruct((G, 1), jnp.float32),
        scratch_shapes=[
            pltpu.VMEM((G, H), jnp.float32),
            pltpu.VMEM((G, 1), jnp.float32),
        ],
        compiler_params=pltpu.CompilerParams(
            dimension_semantics=("arbitrary",)),
    )(hs2, a2, dinv, b2, batchr, Wlin, blin)


def kernel(x, edge_index, batch, W1, b1, W2, b2, Wlin, blin):
    pad = EP - E
    lane = lax.iota(jnp.int32, pad)
    src_p = jnp.concatenate([edge_index[0], lane % 128])
    dst_p = jnp.concatenate([edge_index[1], N + (lane % 16)])
    srcm = src_p.reshape(ER, 128)
    dstm = dst_p.reshape(ER, 128)

    deg2 = _deg_call(dstm)
    dinv, xs = _scale_call(deg2[0, :N, None], deg2[1, :N, None], x)

    o16 = _prop16_call(srcm, dstm, xs)
    hs2 = _mid_call(o16[0, :N], o16[1, :N], xs, dinv, W1,
                    b1.reshape(1, H), W2)

    a2 = _prop128_call(srcm, dstm, hs2)
    out = _final_call(hs2, a2[:N], dinv, b2.reshape(1, H),
                      batch.reshape(100, 1, 1000), Wlin,
                      blin.reshape(1, 1))
    return out[:, 0]


# speedup vs baseline: 15.8884x; 1.8534x over previous
import jax
import jax.numpy as jnp
from jax import lax
from jax.experimental import pallas as pl
from jax.experimental.pallas import tpu as pltpu
from jax.experimental.pallas import tpu_sc as plsc

N = 100000
E = 1600000
F = 16
H = 128
G = 256

NP = 100352
ER = 12544
EP = ER * 128
NCHUNK = 12544
PT = NP // 16

_MESH = dict(core_axis_name="c", subcore_axis_name="s", num_cores=2,
             num_subcores=16)


def _zero_fill(zb):
    rows, cols = zb.shape

    @pl.loop(0, rows)
    def _(i):
        for k in range(cols // 16):
            zb[i, pl.ds(k * 16, 16)] = jnp.zeros((16,), jnp.float32)


def _deg_body(dstm, out, deg_sh, dsts_v, ones_v, zb):
    c = lax.axis_index("c")
    s = lax.axis_index("s")

    @pl.loop(0, 8)
    def _(k):
        ones_v[pl.ds(k * 16, 16)] = jnp.ones((16,), jnp.float32)

    @pl.loop(0, 392)
    def _(i):
        zb[pl.ds(i * 16, 16)] = jnp.zeros((16,), jnp.float32)

    pltpu.sync_copy(zb, deg_sh.at[pl.ds(s * PT, PT)])
    plsc.subcore_barrier()

    @pl.loop(0, 7)
    def _(b):
        rowbase = c * 6272 + s * 392 + b * 56
        pltpu.sync_copy(dstm.at[pl.ds(rowbase, 56), :], dsts_v)

        @pl.loop(0, 56)
        def _(j):
            pltpu.sync_copy(ones_v, deg_sh.at[dsts_v.at[j]], add=True)

    plsc.subcore_barrier()
    pltpu.sync_copy(deg_sh.at[pl.ds(s * PT, PT)],
                    out.at[c, pl.ds(s * PT, PT)])


_deg_call = pl.kernel(
    _deg_body,
    out_type=jax.ShapeDtypeStruct((2, NP), jnp.float32),
    mesh=plsc.VectorSubcoreMesh(**_MESH),
    scratch_types=[
        pltpu.VMEM_SHARED((NP,), jnp.float32),
        pltpu.VMEM((56, 128), jnp.int32),
        pltpu.VMEM((128,), jnp.float32),
        pltpu.VMEM((PT,), jnp.float32),
    ],
)


def _prop16_body(srcm, dstm, xs, out, acc_sh, srcs_v, dsts_v, rows_v, zb):
    c = lax.axis_index("c")
    s = lax.axis_index("s")

    _zero_fill(zb)

    @pl.loop(0, 98)
    def _(k):
        pltpu.sync_copy(zb, acc_sh.at[pl.ds(s * PT + k * 64, 64), :])

    plsc.subcore_barrier()

    @pl.loop(0, 7)
    def _(b):
        rowbase = c * 6272 + s * 392 + b * 56
        pltpu.sync_copy(srcm.at[pl.ds(rowbase, 56), :], srcs_v)
        pltpu.sync_copy(dstm.at[pl.ds(rowbase, 56), :], dsts_v)

        @pl.loop(0, 56)
        def _(j):
            pltpu.sync_copy(xs.at[srcs_v.at[j]], rows_v)
            pltpu.sync_copy(rows_v, acc_sh.at[dsts_v.at[j]], add=True)

    plsc.subcore_barrier()
    pltpu.sync_copy(acc_sh.at[pl.ds(s * PT, PT), :],
                    out.at[c, pl.ds(s * PT, PT), :])


_prop16_call = pl.kernel(
    _prop16_body,
    out_type=jax.ShapeDtypeStruct((2, NP, F), jnp.float32),
    mesh=plsc.VectorSubcoreMesh(**_MESH),
    compiler_params=pltpu.CompilerParams(use_tc_tiling_on_sc=False),
    scratch_types=[
        pltpu.VMEM_SHARED((NP, F), jnp.float32),
        pltpu.VMEM((56, 128), jnp.int32),
        pltpu.VMEM((56, 128), jnp.int32),
        pltpu.VMEM((128, F), jnp.float32),
        pltpu.VMEM((64, F), jnp.float32),
    ],
)


def _prop128_body(srcm, dstm, hs2, out, acc_sh, srcs_v, dsts_v, sf_src,
                  sf_rel, u_src, u_rel, rows_v, zb):
    c = lax.axis_index("c")
    s = lax.axis_index("s")

    _zero_fill(zb)

    for p in range(4):
        base = (2 * p + c) * NCHUNK

        @pl.loop(0, 49)
        def _(k):
            pltpu.sync_copy(zb, acc_sh.at[pl.ds(s * 784 + k * 16, 16), :])

        plsc.subcore_barrier()

        @pl.loop(0, 49)
        def _(b):
            rowbase = s * 784 + b * 16
            pltpu.sync_copy(srcm.at[pl.ds(rowbase, 16), :], srcs_v)
            pltpu.sync_copy(dstm.at[pl.ds(rowbase, 16), :], dsts_v)

            @pl.loop(0, 136)
            def _(i):
                sf_src[pl.ds(i * 16, 16)] = jnp.full((16,), -1, jnp.int32)
                sf_rel[pl.ds(i * 16, 16)] = jnp.full((16,), -1, jnp.int32)

            cl = jnp.int32(0)
            for i in range(128):
                j, k = divmod(i, 8)
                sl = pl.ds(k * 16, 16)
                rel = dsts_v[j, sl] - base
                m = (rel >= 0) & (rel < NCHUNK)
                plsc.store_compressed(sf_rel.at[pl.ds(cl, 16)], rel, mask=m)
                plsc.store_compressed(sf_src.at[pl.ds(cl, 16)],
                                      srcs_v[j, sl], mask=m)
                cl = cl + plsc.all_reduce_population_count(m)[0]

            neg = jnp.full((16,), -1, jnp.int32)
            sf_rel[pl.ds(cl, 16)] = neg
            sf_src[pl.ds(cl, 16)] = neg

            nun = (cl + 127) // 128

            @pl.loop(0, nun)
            def _(u):
                for t in range(8):
                    tsl = pl.ds(t * 16, 16)
                    u_src[tsl] = sf_src[pl.ds(u * 128 + t * 16, 16)]
                    u_rel[tsl] = sf_rel[pl.ds(u * 128 + t * 16, 16)]
                pltpu.sync_copy(
                    hs2.at[plsc.Indices(u_src, ignored_value=-1)], rows_v)
                pltpu.sync_copy(
                    rows_v,
                    acc_sh.at[plsc.Indices(u_rel, ignored_value=-1)],
                    add=True)

        plsc.subcore_barrier()
        pltpu.sync_copy(acc_sh.at[pl.ds(s * 784, 784), :],
                        out.at[pl.ds(base + s * 784, 784), :])
        plsc.subcore_barrier()


_prop128_call = pl.kernel(
    _prop128_body,
    out_type=jax.ShapeDtypeStruct((NP, H), jnp.float32),
    mesh=plsc.VectorSubcoreMesh(**_MESH),
    compiler_params=pltpu.CompilerParams(needs_layout_passes=False),
    scratch_types=[
        pltpu.VMEM_SHARED((NCHUNK, H), jnp.float32),
        pltpu.VMEM((16, 128), jnp.int32),
        pltpu.VMEM((16, 128), jnp.int32),
        pltpu.VMEM((2176,), jnp.int32),
        pltpu.VMEM((2176,), jnp.int32),
        pltpu.VMEM((128,), jnp.int32),
        pltpu.VMEM((128,), jnp.int32),
        pltpu.VMEM((128, H), jnp.float32),
        pltpu.VMEM((16, 128), jnp.float32),
    ],
)


def _scale_body(d0, d1, x, dinv_ref, xs_ref):
    dv = lax.rsqrt(d0[...] + d1[...] + 1.0)
    dinv_ref[...] = dv
    xs_ref[...] = x[...] * dv


def _scale_call(d0, d1, x):
    blk = 2000
    grid = N // blk
    return pl.pallas_call(
        _scale_body,
        grid=(grid,),
        in_specs=[
            pl.BlockSpec((blk, 1), lambda i: (i, 0)),
            pl.BlockSpec((blk, 1), lambda i: (i, 0)),
            pl.BlockSpec((blk, F), lambda i: (i, 0)),
        ],
        out_specs=[
            pl.BlockSpec((blk, 1), lambda i: (i, 0)),
            pl.BlockSpec((blk, F), lambda i: (i, 0)),
        ],
        out_shape=[
            jax.ShapeDtypeStruct((N, 1), jnp.float32),
            jax.ShapeDtypeStruct((N, F), jnp.float32),
        ],
    )(d0, d1, x)


def _mid_body(a0, a1, xs, dinv, W1, b1, W2, hs2_ref):
    ax = (a0[...] + a1[...] + xs[...]) * dinv[...]
    h1 = jnp.maximum(
        jnp.dot(ax, W1[...], preferred_element_type=jnp.float32) + b1[...],
        0.0)
    t2 = jnp.dot(h1, W2[...], preferred_element_type=jnp.float32)
    hs2_ref[...] = t2 * dinv[...]


def _mid_call(a0, a1, xs, dinv, W1, b1, W2):
    blk = 1000
    grid = N // blk
    return pl.pallas_call(
        _mid_body,
        grid=(grid,),
        in_specs=[
            pl.BlockSpec((blk, F), lambda i: (i, 0)),
            pl.BlockSpec((blk, F), lambda i: (i, 0)),
            pl.BlockSpec((blk, F), lambda i: (i, 0)),
            pl.BlockSpec((blk, 1), lambda i: (i, 0)),
            pl.BlockSpec((F, H), lambda i: (0, 0)),
            pl.BlockSpec((1, H), lambda i: (0, 0)),
            pl.BlockSpec((H, H), lambda i: (0, 0)),
        ],
        out_specs=pl.BlockSpec((blk, H), lambda i: (i, 0)),
        out_shape=jax.ShapeDtypeStruct((N, H), jnp.float32),
    )(a0, a1, xs, dinv, W1, b1, W2)


def _final_body(hs2, a2, dinv, b2, batch, Wlin, blin, out_ref, accp, cnt):
    i = pl.program_id(0)

    @pl.when(i == 0)
    def _():
        accp[...] = jnp.zeros_like(accp)
        cnt[...] = jnp.zeros_like(cnt)

    h2 = jnp.maximum((a2[...] + hs2[...]) * dinv[...] + b2[...], 0.0)
    b = jnp.reshape(batch[...], (1, batch.shape[-1]))
    oh = (lax.broadcasted_iota(jnp.int32, (G, b.shape[1]), 0) == b
          ).astype(jnp.float32)
    accp[...] += jnp.dot(oh, h2, preferred_element_type=jnp.float32)
    cnt[...] += jnp.sum(oh, axis=1, keepdims=True)

    @pl.when(i == pl.num_programs(0) - 1)
    def _():
        pooled = accp[...] / jnp.maximum(cnt[...], 1.0)
        out_ref[...] = (
            jnp.dot(pooled, Wlin[...], preferred_element_type=jnp.float32)
            + blin[...])


def _final_call(hs2, a2, dinv, b2, batchr, Wlin, blin):
    blk = 1000
    grid = N // blk
    return pl.pallas_call(
        _final_body,
        grid=(grid,),
        in_specs=[
            pl.BlockSpec((blk, H), lambda i: (i, 0)),
            pl.BlockSpec((blk, H), lambda i: (i, 0)),
            pl.BlockSpec((blk, 1), lambda i: (i, 0)),
            pl.BlockSpec((1, H), lambda i: (0, 0)),
            pl.BlockSpec((1, 1, blk), lambda i: (i, 0, 0)),
            pl.BlockSpec((H, 1), lambda i: (0, 0)),
            pl.BlockSpec((1, 1), lambda i: (0, 0)),
        ],
        out_specs=pl.BlockSpec((G, 1), lambda i: (0, 0)),
        out_shape=jax.ShapeDtypeStruct((G, 1), jnp.float32),
        scratch_shapes=[
            pltpu.VMEM((G, H), jnp.float32),
            pltpu.VMEM((G, 1), jnp.float32),
        ],
        compiler_params=pltpu.CompilerParams(
            dimension_semantics=("arbitrary",)),
    )(hs2, a2, dinv, b2, batchr, Wlin, blin)


def kernel(x, edge_index, batch, W1, b1, W2, b2, Wlin, blin):
    pad = EP - E
    lane = lax.iota(jnp.int32, pad)
    src_p = jnp.concatenate([edge_index[0], lane % 128])
    dst_p = jnp.concatenate([edge_index[1], N + (lane % 16)])
    srcm = src_p.reshape(ER, 128)
    dstm = dst_p.reshape(ER, 128)

    deg2 = _deg_call(dstm)
    dinv, xs = _scale_call(deg2[0, :N, None], deg2[1, :N, None], x)

    o16 = _prop16_call(srcm, dstm, xs)
    hs2 = _mid_call(o16[0, :N], o16[1, :N], xs, dinv, W1,
                    b1.reshape(1, H), W2)

    a2 = _prop128_call(srcm, dstm, hs2)
    out = _final_call(hs2, a2[:N], dinv, b2.reshape(1, H),
                      batch.reshape(100, 1, 1000), Wlin,
                      blin.reshape(1, 1))
    return out[:, 0]


# speedup vs baseline: 17.5719x; 1.1060x over previous
import jax
import jax.numpy as jnp
from jax import lax
from jax.experimental import pallas as pl
from jax.experimental.pallas import tpu as pltpu
from jax.experimental.pallas import tpu_sc as plsc

N = 100000
E = 1600000
F = 16
H = 128
G = 256

NP = 100352
ER = 12544
EP = ER * 128
NCHUNK = 12544
PT = NP // 16

_MESH = dict(core_axis_name="c", subcore_axis_name="s", num_cores=2,
             num_subcores=16)


def _zero_fill(zb):
    rows, cols = zb.shape

    @pl.loop(0, rows)
    def _(i):
        for k in range(cols // 16):
            zb[i, pl.ds(k * 16, 16)] = jnp.zeros((16,), jnp.float32)


def _deg_body(dstm, out, deg_sh, dsts_v, ones_v, zb):
    c = lax.axis_index("c")
    s = lax.axis_index("s")

    @pl.loop(0, 8)
    def _(k):
        ones_v[pl.ds(k * 16, 16)] = jnp.ones((16,), jnp.float32)

    @pl.loop(0, 392)
    def _(i):
        zb[pl.ds(i * 16, 16)] = jnp.zeros((16,), jnp.float32)

    pltpu.sync_copy(zb, deg_sh.at[pl.ds(s * PT, PT)])
    plsc.subcore_barrier()

    @pl.loop(0, 7)
    def _(b):
        rowbase = c * 6272 + s * 392 + b * 56
        pltpu.sync_copy(dstm.at[pl.ds(rowbase, 56), :], dsts_v)

        @pl.loop(0, 56)
        def _(j):
            pltpu.sync_copy(ones_v, deg_sh.at[dsts_v.at[j]], add=True)

    plsc.subcore_barrier()
    pltpu.sync_copy(deg_sh.at[pl.ds(s * PT, PT)],
                    out.at[c, pl.ds(s * PT, PT)])


_deg_call = pl.kernel(
    _deg_body,
    out_type=jax.ShapeDtypeStruct((2, NP), jnp.float32),
    mesh=plsc.VectorSubcoreMesh(**_MESH),
    scratch_types=[
        pltpu.VMEM_SHARED((NP,), jnp.float32),
        pltpu.VMEM((56, 128), jnp.int32),
        pltpu.VMEM((128,), jnp.float32),
        pltpu.VMEM((PT,), jnp.float32),
    ],
)


def _prop16_body(srcm, dstm, xs, out, acc_sh, srcs_v, dsts_v, rows_v, zb,
                 gsem, ssem):
    c = lax.axis_index("c")
    s = lax.axis_index("s")

    _zero_fill(zb)

    @pl.loop(0, 98)
    def _(k):
        pltpu.sync_copy(zb, acc_sh.at[pl.ds(s * PT + k * 64, 64), :])

    plsc.subcore_barrier()

    @pl.loop(0, 7)
    def _(b):
        rowbase = c * 6272 + s * 392 + b * 56
        pltpu.sync_copy(srcm.at[pl.ds(rowbase, 56), :], srcs_v)
        pltpu.sync_copy(dstm.at[pl.ds(rowbase, 56), :], dsts_v)

        def gather(j):
            return pltpu.async_copy(xs.at[srcs_v.at[j]], rows_v.at[j % 2],
                                    gsem)

        def scatter(j):
            return pltpu.async_copy(rows_v.at[j % 2],
                                    acc_sh.at[dsts_v.at[j]], ssem, add=True)

        gd = [None] * 56
        sd = [None] * 56
        for j in range(56):
            if j >= 2:
                sd[j - 2].wait()
            gd[j] = gather(j)
            if j >= 1:
                gd[j - 1].wait()
                sd[j - 1] = scatter(j - 1)
        sd[54].wait()
        gd[55].wait()
        sd[55] = scatter(55)
        sd[55].wait()

    plsc.subcore_barrier()
    pltpu.sync_copy(acc_sh.at[pl.ds(s * PT, PT), :],
                    out.at[c, pl.ds(s * PT, PT), :])


_prop16_call = pl.kernel(
    _prop16_body,
    out_type=jax.ShapeDtypeStruct((2, NP, F), jnp.float32),
    mesh=plsc.VectorSubcoreMesh(**_MESH),
    compiler_params=pltpu.CompilerParams(use_tc_tiling_on_sc=False),
    scratch_types=[
        pltpu.VMEM_SHARED((NP, F), jnp.float32),
        pltpu.VMEM((56, 128), jnp.int32),
        pltpu.VMEM((56, 128), jnp.int32),
        pltpu.VMEM((2, 128, F), jnp.float32),
        pltpu.VMEM((64, F), jnp.float32),
        pltpu.SemaphoreType.DMA,
        pltpu.SemaphoreType.DMA,
    ],
)


def _prop128_body(srcm, dstm, hs2, out, acc_sh, srcs_v, dsts_v, sf_src,
                  sf_rel, u_src, u_rel, rows_v, zb):
    c = lax.axis_index("c")
    s = lax.axis_index("s")

    _zero_fill(zb)

    for p in range(4):
        base = (2 * p + c) * NCHUNK

        @pl.loop(0, 49)
        def _(k):
            pltpu.sync_copy(zb, acc_sh.at[pl.ds(s * 784 + k * 16, 16), :])

        plsc.subcore_barrier()

        @pl.loop(0, 49)
        def _(b):
            rowbase = s * 784 + b * 16
            pltpu.sync_copy(srcm.at[pl.ds(rowbase, 16), :], srcs_v)
            pltpu.sync_copy(dstm.at[pl.ds(rowbase, 16), :], dsts_v)

            cl = jnp.int32(0)
            for i in range(128):
                j, k = divmod(i, 8)
                sl = pl.ds(k * 16, 16)
                rel = dsts_v[j, sl] - base
                m = (rel >= 0) & (rel < NCHUNK)
                plsc.store_compressed(sf_rel.at[pl.ds(cl, 16)], rel, mask=m)
                plsc.store_compressed(sf_src.at[pl.ds(cl, 16)],
                                      srcs_v[j, sl], mask=m)
                cl = cl + plsc.all_reduce_population_count(m)[0]

            neg = jnp.full((16,), -1, jnp.int32)
            for t in range(8):
                sf_rel[pl.ds(cl + t * 16, 16)] = neg
                sf_src[pl.ds(cl + t * 16, 16)] = neg

            nun = (cl + 127) // 128

            @pl.loop(0, nun)
            def _(u):
                for t in range(8):
                    tsl = pl.ds(t * 16, 16)
                    u_src[tsl] = sf_src[pl.ds(u * 128 + t * 16, 16)]
                    u_rel[tsl] = sf_rel[pl.ds(u * 128 + t * 16, 16)]
                pltpu.sync_copy(
                    hs2.at[plsc.Indices(u_src, ignored_value=-1)], rows_v)
                pltpu.sync_copy(
                    rows_v,
                    acc_sh.at[plsc.Indices(u_rel, ignored_value=-1)],
                    add=True)

        plsc.subcore_barrier()
        pltpu.sync_copy(acc_sh.at[pl.ds(s * 784, 784), :],
                        out.at[pl.ds(base + s * 784, 784), :])
        plsc.subcore_barrier()


_prop128_call = pl.kernel(
    _prop128_body,
    out_type=jax.ShapeDtypeStruct((NP, H), jnp.float32),
    mesh=plsc.VectorSubcoreMesh(**_MESH),
    compiler_params=pltpu.CompilerParams(needs_layout_passes=False),
    scratch_types=[
        pltpu.VMEM_SHARED((NCHUNK, H), jnp.float32),
        pltpu.VMEM((16, 128), jnp.int32),
        pltpu.VMEM((16, 128), jnp.int32),
        pltpu.VMEM((2176,), jnp.int32),
        pltpu.VMEM((2176,), jnp.int32),
        pltpu.VMEM((128,), jnp.int32),
        pltpu.VMEM((128,), jnp.int32),
        pltpu.VMEM((128, H), jnp.float32),
        pltpu.VMEM((16, 128), jnp.float32),
    ],
)


def _scale_body(d0, d1, x, dinv_ref, xs_ref):
    dv = lax.rsqrt(d0[...] + d1[...] + 1.0)
    dinv_ref[...] = dv
    xs_ref[...] = x[...] * dv


def _scale_call(d0, d1, x):
    blk = 2000
    grid = N // blk
    return pl.pallas_call(
        _scale_body,
        grid=(grid,),
        in_specs=[
            pl.BlockSpec((blk, 1), lambda i: (i, 0)),
            pl.BlockSpec((blk, 1), lambda i: (i, 0)),
            pl.BlockSpec((blk, F), lambda i: (i, 0)),
        ],
        out_specs=[
            pl.BlockSpec((blk, 1), lambda i: (i, 0)),
            pl.BlockSpec((blk, F), lambda i: (i, 0)),
        ],
        out_shape=[
            jax.ShapeDtypeStruct((N, 1), jnp.float32),
            jax.ShapeDtypeStruct((N, F), jnp.float32),
        ],
    )(d0, d1, x)


def _mid_body(a0, a1, xs, dinv, W1, b1, W2, hs2_ref):
    ax = (a0[...] + a1[...] + xs[...]) * dinv[...]
    h1 = jnp.maximum(
        jnp.dot(ax, W1[...], preferred_element_type=jnp.float32) + b1[...],
        0.0)
    t2 = jnp.dot(h1, W2[...], preferred_element_type=jnp.float32)
    hs2_ref[...] = t2 * dinv[...]


def _mid_call(a0, a1, xs, dinv, W1, b1, W2):
    blk = 1000
    grid = N // blk
    return pl.pallas_call(
        _mid_body,
        grid=(grid,),
        in_specs=[
            pl.BlockSpec((blk, F), lambda i: (i, 0)),
            pl.BlockSpec((blk, F), lambda i: (i, 0)),
            pl.BlockSpec((blk, F), lambda i: (i, 0)),
            pl.BlockSpec((blk, 1), lambda i: (i, 0)),
            pl.BlockSpec((F, H), lambda i: (0, 0)),
            pl.BlockSpec((1, H), lambda i: (0, 0)),
            pl.BlockSpec((H, H), lambda i: (0, 0)),
        ],
        out_specs=pl.BlockSpec((blk, H), lambda i: (i, 0)),
        out_shape=jax.ShapeDtypeStruct((N, H), jnp.float32),
    )(a0, a1, xs, dinv, W1, b1, W2)


def _final_body(hs2, a2, dinv, b2, batch, Wlin, blin, out_ref, accp, cnt):
    i = pl.program_id(0)

    @pl.when(i == 0)
    def _():
        accp[...] = jnp.zeros_like(accp)
        cnt[...] = jnp.zeros_like(cnt)

    h2 = jnp.maximum((a2[...] + hs2[...]) * dinv[...] + b2[...], 0.0)
    b = jnp.reshape(batch[...], (1, batch.shape[-1]))
    oh = (lax.broadcasted_iota(jnp.int32, (G, b.shape[1]), 0) == b
          ).astype(jnp.float32)
    accp[...] += jnp.dot(oh, h2, preferred_element_type=jnp.float32)
    cnt[...] += jnp.sum(oh, axis=1, keepdims=True)

    @pl.when(i == pl.num_programs(0) - 1)
    def _():
        pooled = accp[...] / jnp.maximum(cnt[...], 1.0)
        out_ref[...] = (
            jnp.dot(pooled, Wlin[...], preferred_element_type=jnp.float32)
            + blin[...])


def _final_call(hs2, a2, dinv, b2, batchr, Wlin, blin):
    blk = 1000
    grid = N // blk
    return pl.pallas_call(
        _final_body,
        grid=(grid,),
        in_specs=[
            pl.BlockSpec((blk, H), lambda i: (i, 0)),
            pl.BlockSpec((blk, H), lambda i: (i, 0)),
            pl.BlockSpec((blk, 1), lambda i: (i, 0)),
            pl.BlockSpec((1, H), lambda i: (0, 0)),
            pl.BlockSpec((1, 1, blk), lambda i: (i, 0, 0)),
            pl.BlockSpec((H, 1), lambda i: (0, 0)),
            pl.BlockSpec((1, 1), lambda i: (0, 0)),
        ],
        out_specs=pl.BlockSpec((G, 1), lambda i: (0, 0)),
        out_shape=jax.ShapeDtypeStruct((G, 1), jnp.float32),
        scratch_shapes=[
            pltpu.VMEM((G, H), jnp.float32),
            pltpu.VMEM((G, 1), jnp.float32),
        ],
        compiler_params=pltpu.CompilerParams(
            dimension_semantics=("arbitrary",)),
    )(hs2, a2, dinv, b2, batchr, Wlin, blin)


def kernel(x, edge_index, batch, W1, b1, W2, b2, Wlin, blin):
    pad = EP - E
    lane = lax.iota(jnp.int32, pad)
    src_p = jnp.concatenate([edge_index[0], lane % 128])
    dst_p = jnp.concatenate([edge_index[1], N + (lane % 16)])
    srcm = src_p.reshape(ER, 128)
    dstm = dst_p.reshape(ER, 128)

    deg2 = _deg_call(dstm)
    dinv, xs = _scale_call(deg2[0, :N, None], deg2[1, :N, None], x)

    o16 = _prop16_call(srcm, dstm, xs)
    hs2 = _mid_call(o16[0, :N], o16[1, :N], xs, dinv, W1,
                    b1.reshape(1, H), W2)

    a2 = _prop128_call(srcm, dstm, hs2)
    out = _final_call(hs2, a2[:N], dinv, b2.reshape(1, H),
                      batch.reshape(100, 1, 1000), Wlin,
                      blin.reshape(1, 1))
    return out[:, 0]
